# baseline (speedup 1.0000x reference)
import jax
import jax.numpy as jnp
from jax import lax
from jax.experimental import pallas as pl
import jax.experimental.pallas.tpu as pltpu

D_MODEL = 4096
D_FF = 14336
N_EXPERTS = 8
B = 32
FF_TILE = 512
NT = D_FF // FF_TILE


def _moe_body(x_ref, gates_t_ref, mask_ref, w1_ref, w3_ref, w2_ref, out_ref,
              acc_ref, wgt_ref):
    i = pl.program_id(0)
    xv = x_ref[...]

    @pl.when(i == 0)
    def _gating():
        acc_ref[...] = jnp.zeros_like(acc_ref)
        logits = lax.dot_general(
            xv, gates_t_ref[...], (((1,), (1,)), ((), ())),
            preferred_element_type=jnp.float32)
        ex0 = jnp.max(logits, axis=1, keepdims=True)
        cond0 = (logits == ex0).astype(jnp.float32)
        neg_min = jnp.finfo(jnp.float32).min
        masked = jnp.where(cond0 > 0, neg_min, logits)
        ex1 = jnp.max(masked, axis=1, keepdims=True)
        cond1 = (logits == ex1).astype(jnp.float32)
        pre = 1.0 / (1.0 + jnp.exp(ex1 - ex0))
        c0 = jnp.dot(cond0, mask_ref[...], preferred_element_type=jnp.float32)
        c1 = jnp.dot(cond1, mask_ref[...], preferred_element_type=jnp.float32)
        wgt_ref[...] = c0 * pre - c1 * (pre - 1.0)

    h1 = jnp.dot(xv, w1_ref[...], preferred_element_type=jnp.float32)
    h3 = jnp.dot(xv, w3_ref[...], preferred_element_type=jnp.float32)
    g = (h1 * jax.nn.sigmoid(h1)) * h3
    acc_ref[...] += jnp.dot(g, w2_ref[...], preferred_element_type=jnp.float32)

    @pl.when(i == NT - 1)
    def _finish():
        out_ref[...] = acc_ref[...] * wgt_ref[...]


@jax.jit
def _moe(x2d, gates, w1, w2, w3, expert_mask):
    gates_t = gates.T
    return pl.pallas_call(
        _moe_body,
        grid=(NT,),
        in_specs=[
            pl.BlockSpec((B, D_MODEL), lambda i: (0, 0)),
            pl.BlockSpec((N_EXPERTS, D_MODEL), lambda i: (0, 0)),
            pl.BlockSpec((N_EXPERTS, 1), lambda i: (0, 0)),
            pl.BlockSpec((D_MODEL, FF_TILE), lambda i: (0, i)),
            pl.BlockSpec((D_MODEL, FF_TILE), lambda i: (0, i)),
            pl.BlockSpec((FF_TILE, D_MODEL), lambda i: (i, 0)),
        ],
        out_specs=pl.BlockSpec((B, D_MODEL), lambda i: (0, 0)),
        out_shape=jax.ShapeDtypeStruct((B, D_MODEL), jnp.float32),
        scratch_shapes=[
            pltpu.VMEM((B, D_MODEL), jnp.float32),
            pltpu.VMEM((B, 1), jnp.float32),
        ],
    )(x2d, gates_t, expert_mask, w1, w3, w2)


def kernel(x, gates, w1, w2, w3, expert_mask):
    x2d = x.reshape(B, D_MODEL)
    out = _moe(x2d, gates, w1, w2, w3, expert_mask)
    return out.reshape(1, 1, B, D_MODEL)


# speedup vs baseline: 1.0109x; 1.0109x over previous
import jax
import jax.numpy as jnp
from jax import lax
from jax.experimental import pallas as pl
import jax.experimental.pallas.tpu as pltpu

D_MODEL = 4096
D_FF = 14336
N_EXPERTS = 8
B = 32
FF_TILE = 256
NT = D_FF // FF_TILE


def _moe_body(x_ref, gates_t_ref, mask_ref, w1_ref, w3_ref, w2_ref, out_ref,
              acc_ref, wgt_ref):
    i = pl.program_id(0)
    xv = x_ref[...]

    @pl.when(i == 0)
    def _gating():
        acc_ref[...] = jnp.zeros_like(acc_ref)
        logits = lax.dot_general(
            xv, gates_t_ref[...], (((1,), (1,)), ((), ())),
            preferred_element_type=jnp.float32)
        ex0 = jnp.max(logits, axis=1, keepdims=True)
        cond0 = (logits == ex0).astype(jnp.float32)
        neg_min = jnp.finfo(jnp.float32).min
        masked = jnp.where(cond0 > 0, neg_min, logits)
        ex1 = jnp.max(masked, axis=1, keepdims=True)
        cond1 = (logits == ex1).astype(jnp.float32)
        pre = 1.0 / (1.0 + jnp.exp(ex1 - ex0))
        c0 = jnp.dot(cond0, mask_ref[...], preferred_element_type=jnp.float32)
        c1 = jnp.dot(cond1, mask_ref[...], preferred_element_type=jnp.float32)
        wgt_ref[...] = c0 * pre - c1 * (pre - 1.0)

    h1 = jnp.dot(xv, w1_ref[...], preferred_element_type=jnp.float32)
    h3 = jnp.dot(xv, w3_ref[...], preferred_element_type=jnp.float32)
    g = (h1 * jax.nn.sigmoid(h1)) * h3
    acc_ref[...] += jnp.dot(g, w2_ref[...], preferred_element_type=jnp.float32)

    @pl.when(i == NT - 1)
    def _finish():
        out_ref[...] = acc_ref[...] * wgt_ref[...]


@jax.jit
def _moe(x2d, gates, w1, w2, w3, expert_mask):
    gates_t = gates.T
    return pl.pallas_call(
        _moe_body,
        grid=(NT,),
        in_specs=[
            pl.BlockSpec((B, D_MODEL), lambda i: (0, 0)),
            pl.BlockSpec((N_EXPERTS, D_MODEL), lambda i: (0, 0)),
            pl.BlockSpec((N_EXPERTS, 1), lambda i: (0, 0)),
            pl.BlockSpec((D_MODEL, FF_TILE), lambda i: (0, i)),
            pl.BlockSpec((D_MODEL, FF_TILE), lambda i: (0, i)),
            pl.BlockSpec((FF_TILE, D_MODEL), lambda i: (i, 0)),
        ],
        out_specs=pl.BlockSpec((B, D_MODEL), lambda i: (0, 0)),
        out_shape=jax.ShapeDtypeStruct((B, D_MODEL), jnp.float32),
        scratch_shapes=[
            pltpu.VMEM((B, D_MODEL), jnp.float32),
            pltpu.VMEM((B, 1), jnp.float32),
        ],
    )(x2d, gates_t, expert_mask, w1, w3, w2)


def kernel(x, gates, w1, w2, w3, expert_mask):
    x2d = x.reshape(B, D_MODEL)
    out = _moe(x2d, gates, w1, w2, w3, expert_mask)
    return out.reshape(1, 1, B, D_MODEL)


# speedup vs baseline: 1.0119x; 1.0009x over previous
import jax
import jax.numpy as jnp
from jax import lax
from jax.experimental import pallas as pl
import jax.experimental.pallas.tpu as pltpu

D_MODEL = 4096
D_FF = 14336
N_EXPERTS = 8
B = 32
FF_TILE = 256
NT = D_FF // FF_TILE


def _moe_body(x_ref, gates_t_ref, mask_ref, w1_ref, w3_ref, w2_ref, out_ref,
              wgt_ref):
    i = pl.program_id(0)
    xv = x_ref[...]

    @pl.when(i == 0)
    def _gating():
        out_ref[...] = jnp.zeros_like(out_ref)
        logits = lax.dot_general(
            xv, gates_t_ref[...], (((1,), (1,)), ((), ())),
            preferred_element_type=jnp.float32)
        ex0 = jnp.max(logits, axis=1, keepdims=True)
        cond0 = (logits == ex0).astype(jnp.float32)
        neg_min = jnp.finfo(jnp.float32).min
        masked = jnp.where(cond0 > 0, neg_min, logits)
        ex1 = jnp.max(masked, axis=1, keepdims=True)
        cond1 = (logits == ex1).astype(jnp.float32)
        pre = 1.0 / (1.0 + jnp.exp(ex1 - ex0))
        c0 = jnp.dot(cond0, mask_ref[...], preferred_element_type=jnp.float32)
        c1 = jnp.dot(cond1, mask_ref[...], preferred_element_type=jnp.float32)
        wgt_ref[...] = c0 * pre - c1 * (pre - 1.0)

    h1 = jnp.dot(xv, w1_ref[...], preferred_element_type=jnp.float32)
    h3 = jnp.dot(xv, w3_ref[...], preferred_element_type=jnp.float32)
    g = (h1 * jax.nn.sigmoid(h1)) * h3
    out_ref[...] += jnp.dot(g, w2_ref[...], preferred_element_type=jnp.float32)

    @pl.when(i == NT - 1)
    def _finish():
        out_ref[...] = out_ref[...] * wgt_ref[...]


@jax.jit
def _moe(x2d, gates, w1, w2, w3, expert_mask):
    gates_t = gates.T
    return pl.pallas_call(
        _moe_body,
        grid=(NT,),
        in_specs=[
            pl.BlockSpec((B, D_MODEL), lambda i: (0, 0)),
            pl.BlockSpec((N_EXPERTS, D_MODEL), lambda i: (0, 0)),
            pl.BlockSpec((N_EXPERTS, 1), lambda i: (0, 0)),
            pl.BlockSpec((D_MODEL, FF_TILE), lambda i: (0, i)),
            pl.BlockSpec((D_MODEL, FF_TILE), lambda i: (0, i)),
            pl.BlockSpec((FF_TILE, D_MODEL), lambda i: (i, 0)),
        ],
        out_specs=pl.BlockSpec((B, D_MODEL), lambda i: (0, 0)),
        out_shape=jax.ShapeDtypeStruct((B, D_MODEL), jnp.float32),
        scratch_shapes=[
            pltpu.VMEM((B, 1), jnp.float32),
        ],
    )(x2d, gates_t, expert_mask, w1, w3, w2)


def kernel(x, gates, w1, w2, w3, expert_mask):
    x2d = x.reshape(B, D_MODEL)
    out = _moe(x2d, gates, w1, w2, w3, expert_mask)
    return out.reshape(1, 1, B, D_MODEL)
